# baseline (speedup 1.0000x reference)
import functools

import jax
import jax.numpy as jnp
from jax import lax
from jax.experimental import pallas as pl
from jax.experimental.pallas import tpu as pltpu
from jax.experimental.pallas import tpu_sc as plsc

_TEMP_INV = 5.0
_EPS = 1e-12

_SC_CORES = 2
_SC_SUBCORES = 16
_SC_WORKERS = _SC_CORES * _SC_SUBCORES


def _argmax_body(masked, nb, bchunk, qblk, ids_ref, qids_ref, p_ref, q_ref,
                 out_ref, hi_ref, ties_ref, vmax_ref, vidx_ref):
    qi = pl.program_id(0)

    @pl.when(qi == 0)
    def _init():
        vmax_ref[...] = jnp.full(vmax_ref.shape, -jnp.inf, jnp.float32)
        vidx_ref[...] = jnp.zeros(vidx_ref.shape, jnp.float32)

    q = q_ref[...]
    s2 = jnp.sum(q * q, axis=1, keepdims=True)
    inv = 1.0 / jnp.maximum(jnp.sqrt(s2), _EPS)
    qbf = (q * inv).astype(jnp.bfloat16)
    if masked:
        qids = qids_ref[...]
    iota0 = (lax.broadcasted_iota(jnp.int32, (qblk, bchunk), 0)
             .astype(jnp.float32) + jnp.float32(qi * qblk))

    for b in range(nb):
        pblk = p_ref[b * bchunk:(b + 1) * bchunk]
        simt = lax.dot_general(qbf, pblk, (((1,), (1,)), ((), ())),
                               preferred_element_type=jnp.float32)
        if masked:
            idsb = ids_ref[b:b + 1, :]
            simt = jnp.where(qids == idsb, -jnp.inf, simt)
        lmax = jnp.max(simt, axis=0, keepdims=True)
        larg = jnp.min(jnp.where(simt == lmax, iota0, jnp.float32(1e9)),
                       axis=0, keepdims=True)
        old_v = vmax_ref[b:b + 1, :]
        old_i = vidx_ref[b:b + 1, :]
        upd = lmax > old_v
        vmax_ref[b:b + 1, :] = jnp.where(upd, lmax, old_v)
        vidx_ref[b:b + 1, :] = jnp.where(upd, larg, old_i)

    @pl.when(qi == pl.num_programs(0) - 1)
    def _flush():
        idx = vidx_ref[...].astype(jnp.int32)
        out_ref[...] = idx
        hi_ref[...] = lax.shift_right_logical(idx, 7)
        ties_ref[0] = jnp.int32(0)


def _nn_argmax(p_bf, ids2, queue, qids_col, qblk, bchunk, masked):
    bt, d = p_bf.shape
    q, _ = queue.shape
    nb = bt // bchunk
    nq = q // qblk
    body = functools.partial(_argmax_body, masked, nb, bchunk, qblk)
    return pl.pallas_call(
        body,
        grid=(nq,),
        in_specs=[
            pl.BlockSpec((nb, bchunk), lambda qi: (0, 0)),
            pl.BlockSpec((qblk, 1), lambda qi: (qi, 0)),
            pl.BlockSpec((bt, d), lambda qi: (0, 0)),
            pl.BlockSpec((qblk, d), lambda qi: (qi, 0)),
        ],
        out_specs=[
            pl.BlockSpec((nb, bchunk), lambda qi: (0, 0)),
            pl.BlockSpec((nb, bchunk), lambda qi: (0, 0)),
            pl.BlockSpec(memory_space=pltpu.SMEM),
        ],
        out_shape=[
            jax.ShapeDtypeStruct((nb, bchunk), jnp.int32),
            jax.ShapeDtypeStruct((nb, bchunk), jnp.int32),
            jax.ShapeDtypeStruct((1,), jnp.int32),
        ],
        scratch_shapes=[
            pltpu.VMEM((nb, bchunk), jnp.float32),
            pltpu.VMEM((nb, bchunk), jnp.float32),
        ],
    )(ids2, qids_col, p_bf, queue)


def _argmax_fast_body(nb, bchunk, qblk, p_ref, q_ref, idx_ref, hi_ref,
                      ties_ref, vmax_ref, vidx_ref):
    qi = pl.program_id(0)

    @pl.when(qi == 0)
    def _init():
        vmax_ref[...] = jnp.full(vmax_ref.shape, -jnp.inf, jnp.float32)
        vidx_ref[...] = jnp.zeros(vidx_ref.shape, jnp.float32)

    qq = q_ref[...]
    s2 = jnp.sum(qq * qq, axis=1, keepdims=True)
    inv = 1.0 / jnp.maximum(jnp.sqrt(s2), _EPS)
    qbf = (qq * inv).astype(jnp.bfloat16)
    iota0 = (lax.broadcasted_iota(jnp.int32, (qblk, bchunk), 0)
             .astype(jnp.float32) + jnp.float32(qi * qblk))

    for b in range(nb):
        pblk = p_ref[b * bchunk:(b + 1) * bchunk]
        simt = lax.dot_general(qbf, pblk, (((1,), (1,)), ((), ())),
                               preferred_element_type=jnp.float32)
        lmax = jnp.max(simt, axis=0, keepdims=True)
        larg = jnp.min(jnp.where(simt == lmax, iota0, jnp.float32(1e9)),
                       axis=0, keepdims=True)
        old_v = vmax_ref[b:b + 1, :]
        old_i = vidx_ref[b:b + 1, :]
        upd = lmax > old_v
        vmax_ref[b:b + 1, :] = jnp.where(upd, lmax, old_v)
        vidx_ref[b:b + 1, :] = jnp.where(upd, larg, old_i)

    @pl.when(qi == pl.num_programs(0) - 1)
    def _flush():
        idx = vidx_ref[...].astype(jnp.int32)
        idx_ref[...] = idx
        hi_ref[...] = lax.shift_right_logical(idx, 7)
        ties_ref[0] = jnp.int32(0)


def _nn_argmax_fast(p_bf, queue, qblk, bchunk):
    bt, d = p_bf.shape
    q, _ = queue.shape
    nb = bt // bchunk
    nq = q // qblk
    body = functools.partial(_argmax_fast_body, nb, bchunk, qblk)
    return pl.pallas_call(
        body,
        grid=(nq,),
        in_specs=[
            pl.BlockSpec((bt, d), lambda qi: (0, 0)),
            pl.BlockSpec((qblk, d), lambda qi: (qi, 0)),
        ],
        out_specs=[
            pl.BlockSpec((nb, bchunk), lambda qi: (0, 0)),
            pl.BlockSpec((nb, bchunk), lambda qi: (0, 0)),
            pl.BlockSpec(memory_space=pltpu.SMEM),
        ],
        out_shape=[
            jax.ShapeDtypeStruct((nb, bchunk), jnp.int32),
            jax.ShapeDtypeStruct((nb, bchunk), jnp.int32),
            jax.ShapeDtypeStruct((1,), jnp.int32),
        ],
        scratch_shapes=[
            pltpu.VMEM((nb, bchunk), jnp.float32),
            pltpu.VMEM((nb, bchunk), jnp.float32),
        ],
    )(p_bf, queue)


def _sc_gather(queue, idx2):
    q, d = queue.shape
    nrow, _ = idx2.shape
    bt = nrow * 128
    rows_per_w = nrow // _SC_WORKERS
    mesh = plsc.VectorSubcoreMesh(core_axis_name="c", subcore_axis_name="s")

    @functools.partial(
        pl.kernel,
        mesh=mesh,
        out_type=jax.ShapeDtypeStruct((bt, d), jnp.float32),
        scratch_types=[
            pltpu.VMEM((rows_per_w, 128), jnp.int32),
            pltpu.VMEM((rows_per_w, 128, d), jnp.float32),
            pltpu.SemaphoreType.DMA,
        ],
    )
    def gather(table_hbm, idx_hbm, out_hbm, idx_v, rows_v, sem):
        wid = lax.axis_index("s") * _SC_CORES + lax.axis_index("c")
        base = wid * rows_per_w
        pltpu.sync_copy(idx_hbm.at[pl.ds(base, rows_per_w)], idx_v)
        for j in range(rows_per_w):
            pltpu.async_copy(table_hbm.at[idx_v.at[j]], rows_v.at[j],
                             sem).wait()
        for j in range(rows_per_w):
            pltpu.sync_copy(rows_v.at[j],
                            out_hbm.at[pl.ds((base + j) * 128, 128)])

    return gather(queue, idx2)


def _sc_gather_verify(queue, qid_tbl, idx2, idxhi2):
    q, d = queue.shape
    nrow, _ = idx2.shape
    bt = nrow * 128
    rows_per_w = nrow // _SC_WORKERS
    mesh = plsc.VectorSubcoreMesh(core_axis_name="c", subcore_axis_name="s")

    @functools.partial(
        pl.kernel,
        mesh=mesh,
        out_type=(jax.ShapeDtypeStruct((bt, d), jnp.float32),
                  jax.ShapeDtypeStruct((bt, 128), jnp.int32)),
        scratch_types=[
            pltpu.VMEM((rows_per_w, 128), jnp.int32),
            pltpu.VMEM((rows_per_w, 128), jnp.int32),
            pltpu.VMEM((rows_per_w, 128, d), jnp.float32),
            pltpu.VMEM((rows_per_w, 128, 128), jnp.int32),
            pltpu.SemaphoreType.DMA,
            pltpu.SemaphoreType.DMA,
        ],
    )
    def gather(table_hbm, qtbl_hbm, idx_hbm, idxhi_hbm, out_hbm, gid_hbm,
               idx_v, idxhi_v, rows_v, gids_v, sem_r, sem_q):
        wid = lax.axis_index("s") * _SC_CORES + lax.axis_index("c")
        base = wid * rows_per_w
        pltpu.sync_copy(idx_hbm.at[pl.ds(base, rows_per_w)], idx_v)
        pltpu.sync_copy(idxhi_hbm.at[pl.ds(base, rows_per_w)], idxhi_v)
        for j in range(rows_per_w):
            cr = pltpu.async_copy(table_hbm.at[idx_v.at[j]], rows_v.at[j],
                                  sem_r)
            cq = pltpu.async_copy(qtbl_hbm.at[idxhi_v.at[j]], gids_v.at[j],
                                  sem_q)
            cr.wait()
            cq.wait()
        for j in range(rows_per_w):
            pltpu.sync_copy(rows_v.at[j],
                            out_hbm.at[pl.ds((base + j) * 128, 128)])
            pltpu.sync_copy(gids_v.at[j],
                            gid_hbm.at[pl.ds((base + j) * 128, 128)])

    return gather(queue, qid_tbl, idx2, idxhi2)


def _verify_body(gid_ref, idx_ref, ids_ref, out_ref):
    lane = lax.broadcasted_iota(jnp.int32, gid_ref.shape, 1)
    lo = idx_ref[...] & 127
    picked = jnp.sum(jnp.where(lane == lo, gid_ref[...], 0), axis=1,
                     keepdims=True)
    out_ref[0] = jnp.sum((picked == ids_ref[...]).astype(jnp.int32))


def _count_collisions(gids, idx_col, ids_col):
    bt, _ = gids.shape
    return pl.pallas_call(
        _verify_body,
        in_specs=[
            pl.BlockSpec((bt, 128), lambda: (0, 0)),
            pl.BlockSpec((bt, 1), lambda: (0, 0)),
            pl.BlockSpec((bt, 1), lambda: (0, 0)),
        ],
        out_specs=pl.BlockSpec(memory_space=pltpu.SMEM),
        out_shape=jax.ShapeDtypeStruct((1,), jnp.int32),
    )(gids, idx_col, ids_col)


def _loss_body(nbc, bc, nn_ref, pred_ref, out_ref, acc_ref):
    v = pl.program_id(0)
    b = pl.program_id(1)

    @pl.when(b == 0)
    def _init():
        acc_ref[0, 0] = jnp.float32(0.0)

    pred = pred_ref[0]
    ps2 = jnp.sum(pred * pred, axis=1, keepdims=True)
    predn = (pred / jnp.maximum(jnp.sqrt(ps2), _EPS)).astype(jnp.bfloat16)

    nn = nn_ref[0]
    ns2 = jnp.sum(nn * nn, axis=1, keepdims=True)
    nnn = (nn / jnp.maximum(jnp.sqrt(ns2), _EPS)).astype(jnp.bfloat16)

    logits = lax.dot_general(nnn, predn, (((1,), (1,)), ((), ())),
                             preferred_element_type=jnp.float32) * _TEMP_INV
    m = jnp.max(logits, axis=1, keepdims=True)
    lse = m + jnp.log(jnp.sum(jnp.exp(logits - m), axis=1, keepdims=True))
    rows = lax.broadcasted_iota(jnp.int32, logits.shape, 0) + b * bc
    cols = lax.broadcasted_iota(jnp.int32, logits.shape, 1)
    diag = jnp.sum(jnp.where(rows == cols, logits, 0.0), axis=1, keepdims=True)
    acc_ref[0, 0] += jnp.sum(lse - diag)

    @pl.when(b == nbc - 1)
    def _flush():
        out_ref[v] = acc_ref[0, 0] / (nbc * bc)


def _loss_from_nn(nn, pred_pair, bc):
    _, bsz, d = nn.shape
    nbc = bsz // bc
    body = functools.partial(_loss_body, nbc, bc)
    return pl.pallas_call(
        body,
        grid=(2, nbc),
        in_specs=[
            pl.BlockSpec((1, bc, d), lambda v, b: (v, b, 0)),
            pl.BlockSpec((1, bsz, d), lambda v, b: (v, 0, 0)),
        ],
        out_specs=pl.BlockSpec(memory_space=pltpu.SMEM),
        out_shape=jax.ShapeDtypeStruct((2,), jnp.float32),
        scratch_shapes=[pltpu.SMEM((1, 1), jnp.float32)],
    )(nn, pred_pair)


def kernel(projected, predicted, ids, queue, queue_ids):
    nviews, bsz, d = projected.shape
    q, _ = queue.shape
    bt = nviews * bsz

    qblk = 1024 if q % 1024 == 0 else q
    bchunk = 1024 if bt % 1024 == 0 else bt
    bc = 512 if bsz % 512 == 0 else bsz

    p_all = projected.reshape(bt, d).astype(jnp.bfloat16)
    ids32 = ids.astype(jnp.int32)
    ids_all = jnp.concatenate([ids32] * nviews)
    ids2 = ids_all.reshape(bt // bchunk, bchunk)
    qids32 = queue_ids.astype(jnp.int32)
    qids_col = qids32.reshape(q, 1)
    qid_tbl = qids32.reshape(q // 128, 128)

    idx_u, idxhi_u, ties = _nn_argmax_fast(p_all, queue, qblk, bchunk)
    nn_u, gids = _sc_gather_verify(queue, qid_tbl,
                                   idx_u.reshape(bt // 128, 128),
                                   idxhi_u.reshape(bt // 128, 128))
    nbad = _count_collisions(gids, idx_u.reshape(bt, 1),
                             ids_all.reshape(bt, 1))

    def _exact_path(_):
        idx_m, _unused_hi, _unused_t = _nn_argmax(p_all, ids2, queue,
                                                  qids_col, qblk, bchunk,
                                                  masked=True)
        return _sc_gather(queue, idx_m.reshape(bt // 128, 128))

    nn_flat = lax.cond((nbad[0] > 0) | (ties[0] > 0), _exact_path,
                       lambda _: nn_u, None)
    nn = nn_flat.reshape(nviews, bsz, d)

    pred_pair = jnp.stack([predicted[1], predicted[0]])
    losses = _loss_from_nn(nn, pred_pair, bc)
    return (losses[0], losses[1])


# speedup vs baseline: 1.0411x; 1.0411x over previous
import functools

import jax
import jax.numpy as jnp
from jax import lax
from jax.experimental import pallas as pl
from jax.experimental.pallas import tpu as pltpu
from jax.experimental.pallas import tpu_sc as plsc

_TEMP_INV = 5.0
_EPS = 1e-12

_SC_CORES = 2
_SC_SUBCORES = 16
_SC_WORKERS = _SC_CORES * _SC_SUBCORES


def _argmax_body(masked, nb, bchunk, qblk, ids_ref, qids_ref, p_ref, q_ref,
                 out_ref, hi_ref, ties_ref, vmax_ref, vidx_ref):
    qi = pl.program_id(0)

    @pl.when(qi == 0)
    def _init():
        vmax_ref[...] = jnp.full(vmax_ref.shape, -jnp.inf, jnp.float32)
        vidx_ref[...] = jnp.zeros(vidx_ref.shape, jnp.float32)

    q = q_ref[...]
    s2 = jnp.sum(q * q, axis=1, keepdims=True)
    inv = 1.0 / jnp.maximum(jnp.sqrt(s2), _EPS)
    qbf = (q * inv).astype(jnp.bfloat16)
    if masked:
        qids = qids_ref[...]
    iota0 = (lax.broadcasted_iota(jnp.int32, (qblk, bchunk), 0)
             .astype(jnp.float32) + jnp.float32(qi * qblk))

    for b in range(nb):
        pblk = p_ref[b * bchunk:(b + 1) * bchunk]
        simt = lax.dot_general(qbf, pblk, (((1,), (1,)), ((), ())),
                               preferred_element_type=jnp.float32)
        if masked:
            idsb = ids_ref[b:b + 1, :]
            simt = jnp.where(qids == idsb, -jnp.inf, simt)
        lmax = jnp.max(simt, axis=0, keepdims=True)
        larg = jnp.min(jnp.where(simt == lmax, iota0, jnp.float32(1e9)),
                       axis=0, keepdims=True)
        old_v = vmax_ref[b:b + 1, :]
        old_i = vidx_ref[b:b + 1, :]
        upd = lmax > old_v
        vmax_ref[b:b + 1, :] = jnp.where(upd, lmax, old_v)
        vidx_ref[b:b + 1, :] = jnp.where(upd, larg, old_i)

    @pl.when(qi == pl.num_programs(0) - 1)
    def _flush():
        idx = vidx_ref[...].astype(jnp.int32)
        out_ref[...] = idx
        hi_ref[...] = lax.shift_right_logical(idx, 7)
        ties_ref[0] = jnp.int32(0)


def _nn_argmax(p_bf, ids2, queue, qids_col, qblk, bchunk, masked):
    bt, d = p_bf.shape
    q, _ = queue.shape
    nb = bt // bchunk
    nq = q // qblk
    body = functools.partial(_argmax_body, masked, nb, bchunk, qblk)
    return pl.pallas_call(
        body,
        grid=(nq,),
        in_specs=[
            pl.BlockSpec((nb, bchunk), lambda qi: (0, 0)),
            pl.BlockSpec((qblk, 1), lambda qi: (qi, 0)),
            pl.BlockSpec((bt, d), lambda qi: (0, 0)),
            pl.BlockSpec((qblk, d), lambda qi: (qi, 0)),
        ],
        out_specs=[
            pl.BlockSpec((nb, bchunk), lambda qi: (0, 0)),
            pl.BlockSpec((nb, bchunk), lambda qi: (0, 0)),
            pl.BlockSpec(memory_space=pltpu.SMEM),
        ],
        out_shape=[
            jax.ShapeDtypeStruct((nb, bchunk), jnp.int32),
            jax.ShapeDtypeStruct((nb, bchunk), jnp.int32),
            jax.ShapeDtypeStruct((1,), jnp.int32),
        ],
        scratch_shapes=[
            pltpu.VMEM((nb, bchunk), jnp.float32),
            pltpu.VMEM((nb, bchunk), jnp.float32),
        ],
    )(ids2, qids_col, p_bf, queue)


def _argmax_fast_body(nb, bchunk, qblk, p_ref, q_ref, idx_ref, hi_ref,
                      ties_ref, vmax_ref, vidx_ref):
    qi = pl.program_id(0)

    @pl.when(qi == 0)
    def _init():
        vmax_ref[...] = jnp.full(vmax_ref.shape, -jnp.inf, jnp.float32)
        vidx_ref[...] = jnp.zeros(vidx_ref.shape, jnp.float32)

    qq = q_ref[...]
    s2 = jnp.sum(qq * qq, axis=1, keepdims=True)
    inv = 1.0 / jnp.maximum(jnp.sqrt(s2), _EPS)
    qbf = (qq * inv).astype(jnp.bfloat16)
    iota0 = (lax.broadcasted_iota(jnp.int32, (qblk, bchunk), 0)
             .astype(jnp.float32) + jnp.float32(qi * qblk))

    for b in range(nb):
        pblk = p_ref[b * bchunk:(b + 1) * bchunk]
        simt = lax.dot_general(qbf, pblk, (((1,), (1,)), ((), ())),
                               preferred_element_type=jnp.float32)
        lmax = jnp.max(simt, axis=0, keepdims=True)
        larg = jnp.min(jnp.where(simt == lmax, iota0, jnp.float32(1e9)),
                       axis=0, keepdims=True)
        old_v = vmax_ref[b:b + 1, :]
        old_i = vidx_ref[b:b + 1, :]
        upd = lmax > old_v
        vmax_ref[b:b + 1, :] = jnp.where(upd, lmax, old_v)
        vidx_ref[b:b + 1, :] = jnp.where(upd, larg, old_i)

    @pl.when(qi == pl.num_programs(0) - 1)
    def _flush():
        idx = vidx_ref[...].astype(jnp.int32)
        idx_ref[...] = idx
        hi_ref[...] = lax.shift_right_logical(idx, 7)
        ties_ref[0] = jnp.int32(0)


def _nn_argmax_fast(p_bf, queue, qblk, bchunk):
    bt, d = p_bf.shape
    q, _ = queue.shape
    nb = bt // bchunk
    nq = q // qblk
    body = functools.partial(_argmax_fast_body, nb, bchunk, qblk)
    return pl.pallas_call(
        body,
        grid=(nq,),
        in_specs=[
            pl.BlockSpec((bt, d), lambda qi: (0, 0)),
            pl.BlockSpec((qblk, d), lambda qi: (qi, 0)),
        ],
        out_specs=[
            pl.BlockSpec((nb, bchunk), lambda qi: (0, 0)),
            pl.BlockSpec((nb, bchunk), lambda qi: (0, 0)),
            pl.BlockSpec(memory_space=pltpu.SMEM),
        ],
        out_shape=[
            jax.ShapeDtypeStruct((nb, bchunk), jnp.int32),
            jax.ShapeDtypeStruct((nb, bchunk), jnp.int32),
            jax.ShapeDtypeStruct((1,), jnp.int32),
        ],
        scratch_shapes=[
            pltpu.VMEM((nb, bchunk), jnp.float32),
            pltpu.VMEM((nb, bchunk), jnp.float32),
        ],
    )(p_bf, queue)


def _sc_gather(queue, idx2):
    q, d = queue.shape
    nrow, _ = idx2.shape
    bt = nrow * 128
    rows_per_w = nrow // _SC_WORKERS
    mesh = plsc.VectorSubcoreMesh(core_axis_name="c", subcore_axis_name="s")

    @functools.partial(
        pl.kernel,
        mesh=mesh,
        out_type=jax.ShapeDtypeStruct((bt, d), jnp.float32),
        scratch_types=[
            pltpu.VMEM((rows_per_w, 128), jnp.int32),
            pltpu.VMEM((rows_per_w, 128, d), jnp.float32),
            pltpu.SemaphoreType.DMA,
        ],
    )
    def gather(table_hbm, idx_hbm, out_hbm, idx_v, rows_v, sem):
        wid = lax.axis_index("s") * _SC_CORES + lax.axis_index("c")
        base = wid * rows_per_w
        pltpu.sync_copy(idx_hbm.at[pl.ds(base, rows_per_w)], idx_v)
        for j in range(rows_per_w):
            pltpu.async_copy(table_hbm.at[idx_v.at[j]], rows_v.at[j],
                             sem).wait()
        for j in range(rows_per_w):
            pltpu.sync_copy(rows_v.at[j],
                            out_hbm.at[pl.ds((base + j) * 128, 128)])

    return gather(queue, idx2)


def _sc_gather_verify(queue, qid_tbl, idx2, idxhi2):
    q, d = queue.shape
    nrow, _ = idx2.shape
    bt = nrow * 128
    rows_per_w = nrow // _SC_WORKERS
    mesh = plsc.VectorSubcoreMesh(core_axis_name="c", subcore_axis_name="s")

    @functools.partial(
        pl.kernel,
        mesh=mesh,
        out_type=(jax.ShapeDtypeStruct((bt, d), jnp.float32),
                  jax.ShapeDtypeStruct((bt, 128), jnp.int32)),
        scratch_types=[
            pltpu.VMEM((rows_per_w, 128), jnp.int32),
            pltpu.VMEM((rows_per_w, 128), jnp.int32),
            pltpu.VMEM((rows_per_w, 128, d), jnp.float32),
            pltpu.VMEM((rows_per_w, 128, 128), jnp.int32),
            pltpu.SemaphoreType.DMA,
            pltpu.SemaphoreType.DMA,
        ],
    )
    def gather(table_hbm, qtbl_hbm, idx_hbm, idxhi_hbm, out_hbm, gid_hbm,
               idx_v, idxhi_v, rows_v, gids_v, sem_r, sem_q):
        wid = lax.axis_index("s") * _SC_CORES + lax.axis_index("c")
        base = wid * rows_per_w
        pltpu.sync_copy(idx_hbm.at[pl.ds(base, rows_per_w)], idx_v)
        pltpu.sync_copy(idxhi_hbm.at[pl.ds(base, rows_per_w)], idxhi_v)
        for j in range(rows_per_w):
            cr = pltpu.async_copy(table_hbm.at[idx_v.at[j]], rows_v.at[j],
                                  sem_r)
            cq = pltpu.async_copy(qtbl_hbm.at[idxhi_v.at[j]], gids_v.at[j],
                                  sem_q)
            cr.wait()
            cq.wait()
        for j in range(rows_per_w):
            pltpu.sync_copy(rows_v.at[j],
                            out_hbm.at[pl.ds((base + j) * 128, 128)])
            pltpu.sync_copy(gids_v.at[j],
                            gid_hbm.at[pl.ds((base + j) * 128, 128)])

    return gather(queue, qid_tbl, idx2, idxhi2)


def _verify_body(gid_ref, idx_ref, ids_ref, out_ref):
    lane = lax.broadcasted_iota(jnp.int32, gid_ref.shape, 1)
    lo = idx_ref[...] & 127
    picked = jnp.sum(jnp.where(lane == lo, gid_ref[...], 0), axis=1,
                     keepdims=True)
    out_ref[0] = jnp.sum((picked == ids_ref[...]).astype(jnp.int32))


def _count_collisions(gids, idx_col, ids_col):
    bt, _ = gids.shape
    return pl.pallas_call(
        _verify_body,
        in_specs=[
            pl.BlockSpec((bt, 128), lambda: (0, 0)),
            pl.BlockSpec((bt, 1), lambda: (0, 0)),
            pl.BlockSpec((bt, 1), lambda: (0, 0)),
        ],
        out_specs=pl.BlockSpec(memory_space=pltpu.SMEM),
        out_shape=jax.ShapeDtypeStruct((1,), jnp.int32),
    )(gids, idx_col, ids_col)


def _loss_body(nbc, bc, nn_ref, pred_ref, out_ref, acc_ref):
    v = pl.program_id(0)
    b = pl.program_id(1)

    @pl.when(b == 0)
    def _init():
        acc_ref[0, 0] = jnp.float32(0.0)

    pred = pred_ref[0]
    ps2 = jnp.sum(pred * pred, axis=1, keepdims=True)
    predn = (pred / jnp.maximum(jnp.sqrt(ps2), _EPS)).astype(jnp.bfloat16)

    nn = nn_ref[0]
    ns2 = jnp.sum(nn * nn, axis=1, keepdims=True)
    nnn = (nn / jnp.maximum(jnp.sqrt(ns2), _EPS)).astype(jnp.bfloat16)

    logits = lax.dot_general(nnn, predn, (((1,), (1,)), ((), ())),
                             preferred_element_type=jnp.float32) * _TEMP_INV
    m = jnp.max(logits, axis=1, keepdims=True)
    lse = m + jnp.log(jnp.sum(jnp.exp(logits - m), axis=1, keepdims=True))
    rows = lax.broadcasted_iota(jnp.int32, logits.shape, 0) + b * bc
    cols = lax.broadcasted_iota(jnp.int32, logits.shape, 1)
    diag = jnp.sum(jnp.where(rows == cols, logits, 0.0), axis=1, keepdims=True)
    acc_ref[0, 0] += jnp.sum(lse - diag)

    @pl.when(b == nbc - 1)
    def _flush():
        out_ref[v] = acc_ref[0, 0] / (nbc * bc)


def _loss_from_nn(nn, pred_pair, bc):
    _, bsz, d = nn.shape
    nbc = bsz // bc
    body = functools.partial(_loss_body, nbc, bc)
    return pl.pallas_call(
        body,
        grid=(2, nbc),
        in_specs=[
            pl.BlockSpec((1, bc, d), lambda v, b: (v, b, 0)),
            pl.BlockSpec((1, bsz, d), lambda v, b: (v, 0, 0)),
        ],
        out_specs=pl.BlockSpec(memory_space=pltpu.SMEM),
        out_shape=jax.ShapeDtypeStruct((2,), jnp.float32),
        scratch_shapes=[pltpu.SMEM((1, 1), jnp.float32)],
    )(nn, pred_pair)


def kernel(projected, predicted, ids, queue, queue_ids):
    nviews, bsz, d = projected.shape
    q, _ = queue.shape
    bt = nviews * bsz

    qblk = 2048 if q % 2048 == 0 else (1024 if q % 1024 == 0 else q)
    bchunk = 512 if bt % 512 == 0 else bt
    bc = 512 if bsz % 512 == 0 else bsz

    p_all = projected.reshape(bt, d).astype(jnp.bfloat16)
    ids32 = ids.astype(jnp.int32)
    ids_all = jnp.concatenate([ids32] * nviews)
    ids2 = ids_all.reshape(bt // bchunk, bchunk)
    qids32 = queue_ids.astype(jnp.int32)
    qids_col = qids32.reshape(q, 1)
    qid_tbl = qids32.reshape(q // 128, 128)

    idx_u, idxhi_u, ties = _nn_argmax_fast(p_all, queue, qblk, bchunk)
    nn_u, gids = _sc_gather_verify(queue, qid_tbl,
                                   idx_u.reshape(bt // 128, 128),
                                   idxhi_u.reshape(bt // 128, 128))
    nbad = _count_collisions(gids, idx_u.reshape(bt, 1),
                             ids_all.reshape(bt, 1))

    def _exact_path(_):
        idx_m, _unused_hi, _unused_t = _nn_argmax(p_all, ids2, queue,
                                                  qids_col, qblk, bchunk,
                                                  masked=True)
        return _sc_gather(queue, idx_m.reshape(bt // 128, 128))

    nn_flat = lax.cond((nbad[0] > 0) | (ties[0] > 0), _exact_path,
                       lambda _: nn_u, None)
    nn = nn_flat.reshape(nviews, bsz, d)

    pred_pair = jnp.stack([predicted[1], predicted[0]])
    losses = _loss_from_nn(nn, pred_pair, bc)
    return (losses[0], losses[1])


# speedup vs baseline: 1.0423x; 1.0012x over previous
import functools

import jax
import jax.numpy as jnp
from jax import lax
from jax.experimental import pallas as pl
from jax.experimental.pallas import tpu as pltpu
from jax.experimental.pallas import tpu_sc as plsc

_TEMP_INV = 5.0
_EPS = 1e-12

_SC_CORES = 2
_SC_SUBCORES = 16
_SC_WORKERS = _SC_CORES * _SC_SUBCORES


def _argmax_body(masked, nb, bchunk, qblk, ids_ref, qids_ref, p_ref, q_ref,
                 out_ref, hi_ref, ties_ref, vmax_ref, vidx_ref):
    qi = pl.program_id(0)

    @pl.when(qi == 0)
    def _init():
        vmax_ref[...] = jnp.full(vmax_ref.shape, -jnp.inf, jnp.float32)
        vidx_ref[...] = jnp.zeros(vidx_ref.shape, jnp.float32)

    q = q_ref[...]
    s2 = jnp.sum(q * q, axis=1, keepdims=True)
    inv = 1.0 / jnp.maximum(jnp.sqrt(s2), _EPS)
    qbf = (q * inv).astype(jnp.bfloat16)
    if masked:
        qids = qids_ref[...]
    iota0 = (lax.broadcasted_iota(jnp.int32, (qblk, bchunk), 0)
             .astype(jnp.float32) + jnp.float32(qi * qblk))

    for b in range(nb):
        pblk = p_ref[b * bchunk:(b + 1) * bchunk]
        simt = lax.dot_general(qbf, pblk, (((1,), (1,)), ((), ())),
                               preferred_element_type=jnp.float32)
        if masked:
            idsb = ids_ref[b:b + 1, :]
            simt = jnp.where(qids == idsb, -jnp.inf, simt)
        lmax = jnp.max(simt, axis=0, keepdims=True)
        larg = jnp.min(jnp.where(simt == lmax, iota0, jnp.float32(1e9)),
                       axis=0, keepdims=True)
        old_v = vmax_ref[b:b + 1, :]
        old_i = vidx_ref[b:b + 1, :]
        upd = lmax > old_v
        vmax_ref[b:b + 1, :] = jnp.where(upd, lmax, old_v)
        vidx_ref[b:b + 1, :] = jnp.where(upd, larg, old_i)

    @pl.when(qi == pl.num_programs(0) - 1)
    def _flush():
        idx = vidx_ref[...].astype(jnp.int32)
        out_ref[...] = idx
        hi_ref[...] = lax.shift_right_logical(idx, 7)
        ties_ref[0] = jnp.int32(0)


def _nn_argmax(p_bf, ids2, queue, qids_col, qblk, bchunk, masked):
    bt, d = p_bf.shape
    q, _ = queue.shape
    nb = bt // bchunk
    nq = q // qblk
    body = functools.partial(_argmax_body, masked, nb, bchunk, qblk)
    return pl.pallas_call(
        body,
        grid=(nq,),
        in_specs=[
            pl.BlockSpec((nb, bchunk), lambda qi: (0, 0)),
            pl.BlockSpec((qblk, 1), lambda qi: (qi, 0)),
            pl.BlockSpec((bt, d), lambda qi: (0, 0)),
            pl.BlockSpec((qblk, d), lambda qi: (qi, 0)),
        ],
        out_specs=[
            pl.BlockSpec((nb, bchunk), lambda qi: (0, 0)),
            pl.BlockSpec((nb, bchunk), lambda qi: (0, 0)),
            pl.BlockSpec(memory_space=pltpu.SMEM),
        ],
        out_shape=[
            jax.ShapeDtypeStruct((nb, bchunk), jnp.int32),
            jax.ShapeDtypeStruct((nb, bchunk), jnp.int32),
            jax.ShapeDtypeStruct((1,), jnp.int32),
        ],
        scratch_shapes=[
            pltpu.VMEM((nb, bchunk), jnp.float32),
            pltpu.VMEM((nb, bchunk), jnp.float32),
        ],
    )(ids2, qids_col, p_bf, queue)


def _argmax_fast_body(nb, bchunk, qblk, p_ref, q_ref, idx_ref, hi_ref,
                      ties_ref, vmax_ref, vidx_ref):
    qi = pl.program_id(0)

    @pl.when(qi == 0)
    def _init():
        vmax_ref[...] = jnp.full(vmax_ref.shape, -jnp.inf, jnp.float32)
        vidx_ref[...] = jnp.zeros(vidx_ref.shape, jnp.float32)

    qq = q_ref[...]
    s2 = jnp.sum(qq * qq, axis=1, keepdims=True)
    inv = 1.0 / jnp.maximum(jnp.sqrt(s2), _EPS)
    qbf = (qq * inv).astype(jnp.bfloat16)
    iota0 = (lax.broadcasted_iota(jnp.int32, (qblk, bchunk), 0)
             .astype(jnp.float32) + jnp.float32(qi * qblk))

    for b in range(nb):
        pblk = p_ref[b * bchunk:(b + 1) * bchunk]
        simt = lax.dot_general(qbf, pblk, (((1,), (1,)), ((), ())),
                               preferred_element_type=jnp.float32)
        lmax = jnp.max(simt, axis=0, keepdims=True)
        larg = jnp.min(jnp.where(simt == lmax, iota0, jnp.float32(1e9)),
                       axis=0, keepdims=True)
        old_v = vmax_ref[b:b + 1, :]
        old_i = vidx_ref[b:b + 1, :]
        upd = lmax > old_v
        vmax_ref[b:b + 1, :] = jnp.where(upd, lmax, old_v)
        vidx_ref[b:b + 1, :] = jnp.where(upd, larg, old_i)

    @pl.when(qi == pl.num_programs(0) - 1)
    def _flush():
        idx = vidx_ref[...].astype(jnp.int32)
        idx_ref[...] = idx
        hi_ref[...] = lax.shift_right_logical(idx, 7)
        ties_ref[0] = jnp.int32(0)


def _nn_argmax_fast(p_bf, queue, qblk, bchunk):
    bt, d = p_bf.shape
    q, _ = queue.shape
    nb = bt // bchunk
    nq = q // qblk
    body = functools.partial(_argmax_fast_body, nb, bchunk, qblk)
    return pl.pallas_call(
        body,
        grid=(nq,),
        in_specs=[
            pl.BlockSpec((bt, d), lambda qi: (0, 0)),
            pl.BlockSpec((qblk, d), lambda qi: (qi, 0)),
        ],
        out_specs=[
            pl.BlockSpec((nb, bchunk), lambda qi: (0, 0)),
            pl.BlockSpec((nb, bchunk), lambda qi: (0, 0)),
            pl.BlockSpec(memory_space=pltpu.SMEM),
        ],
        out_shape=[
            jax.ShapeDtypeStruct((nb, bchunk), jnp.int32),
            jax.ShapeDtypeStruct((nb, bchunk), jnp.int32),
            jax.ShapeDtypeStruct((1,), jnp.int32),
        ],
        scratch_shapes=[
            pltpu.VMEM((nb, bchunk), jnp.float32),
            pltpu.VMEM((nb, bchunk), jnp.float32),
        ],
    )(p_bf, queue)


def _sc_gather(queue, idx2):
    q, d = queue.shape
    nrow, _ = idx2.shape
    bt = nrow * 128
    rows_per_w = nrow // _SC_WORKERS
    mesh = plsc.VectorSubcoreMesh(core_axis_name="c", subcore_axis_name="s")

    @functools.partial(
        pl.kernel,
        mesh=mesh,
        out_type=jax.ShapeDtypeStruct((bt, d), jnp.float32),
        scratch_types=[
            pltpu.VMEM((rows_per_w, 128), jnp.int32),
            pltpu.VMEM((rows_per_w, 128, d), jnp.float32),
            pltpu.SemaphoreType.DMA,
        ],
    )
    def gather(table_hbm, idx_hbm, out_hbm, idx_v, rows_v, sem):
        wid = lax.axis_index("s") * _SC_CORES + lax.axis_index("c")
        base = wid * rows_per_w
        pltpu.sync_copy(idx_hbm.at[pl.ds(base, rows_per_w)], idx_v)
        for j in range(rows_per_w):
            pltpu.async_copy(table_hbm.at[idx_v.at[j]], rows_v.at[j],
                             sem).wait()
        for j in range(rows_per_w):
            pltpu.sync_copy(rows_v.at[j],
                            out_hbm.at[pl.ds((base + j) * 128, 128)])

    return gather(queue, idx2)


def _sc_gather_verify(queue, qid_tbl, idx2, idxhi2):
    q, d = queue.shape
    nrow, _ = idx2.shape
    bt = nrow * 128
    rows_per_w = nrow // _SC_WORKERS
    mesh = plsc.VectorSubcoreMesh(core_axis_name="c", subcore_axis_name="s")

    @functools.partial(
        pl.kernel,
        mesh=mesh,
        out_type=(jax.ShapeDtypeStruct((bt, d), jnp.float32),
                  jax.ShapeDtypeStruct((bt, 128), jnp.int32)),
        scratch_types=[
            pltpu.VMEM((rows_per_w, 128), jnp.int32),
            pltpu.VMEM((rows_per_w, 128), jnp.int32),
            pltpu.VMEM((rows_per_w, 128, d), jnp.float32),
            pltpu.VMEM((rows_per_w, 128, 128), jnp.int32),
            pltpu.SemaphoreType.DMA,
            pltpu.SemaphoreType.DMA,
        ],
    )
    def gather(table_hbm, qtbl_hbm, idx_hbm, idxhi_hbm, out_hbm, gid_hbm,
               idx_v, idxhi_v, rows_v, gids_v, sem_r, sem_q):
        wid = lax.axis_index("s") * _SC_CORES + lax.axis_index("c")
        base = wid * rows_per_w
        pltpu.sync_copy(idx_hbm.at[pl.ds(base, rows_per_w)], idx_v)
        pltpu.sync_copy(idxhi_hbm.at[pl.ds(base, rows_per_w)], idxhi_v)
        for j in range(rows_per_w):
            cr = pltpu.async_copy(table_hbm.at[idx_v.at[j]], rows_v.at[j],
                                  sem_r)
            cq = pltpu.async_copy(qtbl_hbm.at[idxhi_v.at[j]], gids_v.at[j],
                                  sem_q)
            cr.wait()
            cq.wait()
        for j in range(rows_per_w):
            pltpu.sync_copy(rows_v.at[j],
                            out_hbm.at[pl.ds((base + j) * 128, 128)])
            pltpu.sync_copy(gids_v.at[j],
                            gid_hbm.at[pl.ds((base + j) * 128, 128)])

    return gather(queue, qid_tbl, idx2, idxhi2)


def _verify_body(gid_ref, idx_ref, ids_ref, out_ref):
    lane = lax.broadcasted_iota(jnp.int32, gid_ref.shape, 1)
    lo = idx_ref[...] & 127
    picked = jnp.sum(jnp.where(lane == lo, gid_ref[...], 0), axis=1,
                     keepdims=True)
    out_ref[0] = jnp.sum((picked == ids_ref[...]).astype(jnp.int32))


def _count_collisions(gids, idx_col, ids_col):
    bt, _ = gids.shape
    return pl.pallas_call(
        _verify_body,
        in_specs=[
            pl.BlockSpec((bt, 128), lambda: (0, 0)),
            pl.BlockSpec((bt, 1), lambda: (0, 0)),
            pl.BlockSpec((bt, 1), lambda: (0, 0)),
        ],
        out_specs=pl.BlockSpec(memory_space=pltpu.SMEM),
        out_shape=jax.ShapeDtypeStruct((1,), jnp.int32),
    )(gids, idx_col, ids_col)


def _loss_body(nbc, bc, nn_ref, pred_ref, out_ref, acc_ref):
    v = pl.program_id(0)
    b = pl.program_id(1)

    @pl.when(b == 0)
    def _init():
        acc_ref[0, 0] = jnp.float32(0.0)

    pred = pred_ref[0]
    ps2 = jnp.sum(pred * pred, axis=1, keepdims=True)
    predn = (pred / jnp.maximum(jnp.sqrt(ps2), _EPS)).astype(jnp.bfloat16)

    nn = nn_ref[0]
    ns2 = jnp.sum(nn * nn, axis=1, keepdims=True)
    nnn = (nn / jnp.maximum(jnp.sqrt(ns2), _EPS)).astype(jnp.bfloat16)

    logits = lax.dot_general(nnn, predn, (((1,), (1,)), ((), ())),
                             preferred_element_type=jnp.float32) * _TEMP_INV
    m = jnp.max(logits, axis=1, keepdims=True)
    lse = m + jnp.log(jnp.sum(jnp.exp(logits - m), axis=1, keepdims=True))
    rows = lax.broadcasted_iota(jnp.int32, logits.shape, 0) + b * bc
    cols = lax.broadcasted_iota(jnp.int32, logits.shape, 1)
    diag = jnp.sum(jnp.where(rows == cols, logits, 0.0), axis=1, keepdims=True)
    acc_ref[0, 0] += jnp.sum(lse - diag)

    @pl.when(b == nbc - 1)
    def _flush():
        out_ref[v] = acc_ref[0, 0] / (nbc * bc)


def _loss_from_nn(nn, pred_pair, bc):
    _, bsz, d = nn.shape
    nbc = bsz // bc
    body = functools.partial(_loss_body, nbc, bc)
    return pl.pallas_call(
        body,
        grid=(2, nbc),
        in_specs=[
            pl.BlockSpec((1, bc, d), lambda v, b: (v, b, 0)),
            pl.BlockSpec((1, bsz, d), lambda v, b: (v, 0, 0)),
        ],
        out_specs=pl.BlockSpec(memory_space=pltpu.SMEM),
        out_shape=jax.ShapeDtypeStruct((2,), jnp.float32),
        scratch_shapes=[pltpu.SMEM((1, 1), jnp.float32)],
    )(nn, pred_pair)


def kernel(projected, predicted, ids, queue, queue_ids):
    nviews, bsz, d = projected.shape
    q, _ = queue.shape
    bt = nviews * bsz

    qblk = 4096 if q % 4096 == 0 else (1024 if q % 1024 == 0 else q)
    bchunk = 512 if bt % 512 == 0 else bt
    bc = 512 if bsz % 512 == 0 else bsz

    p_all = projected.reshape(bt, d).astype(jnp.bfloat16)
    ids32 = ids.astype(jnp.int32)
    ids_all = jnp.concatenate([ids32] * nviews)
    ids2 = ids_all.reshape(bt // bchunk, bchunk)
    qids32 = queue_ids.astype(jnp.int32)
    qids_col = qids32.reshape(q, 1)
    qid_tbl = qids32.reshape(q // 128, 128)

    idx_u, idxhi_u, ties = _nn_argmax_fast(p_all, queue, qblk, bchunk)
    nn_u, gids = _sc_gather_verify(queue, qid_tbl,
                                   idx_u.reshape(bt // 128, 128),
                                   idxhi_u.reshape(bt // 128, 128))
    nbad = _count_collisions(gids, idx_u.reshape(bt, 1),
                             ids_all.reshape(bt, 1))

    def _exact_path(_):
        idx_m, _unused_hi, _unused_t = _nn_argmax(p_all, ids2, queue,
                                                  qids_col, qblk, bchunk,
                                                  masked=True)
        return _sc_gather(queue, idx_m.reshape(bt // 128, 128))

    nn_flat = lax.cond((nbad[0] > 0) | (ties[0] > 0), _exact_path,
                       lambda _: nn_u, None)
    nn = nn_flat.reshape(nviews, bsz, d)

    pred_pair = jnp.stack([predicted[1], predicted[0]])
    losses = _loss_from_nn(nn, pred_pair, bc)
    return (losses[0], losses[1])
